# baseline (speedup 1.0000x reference)
import jax
import jax.numpy as jnp
from jax import lax
from jax.experimental import pallas as pl
from jax.experimental.pallas import tpu as pltpu
from jax.experimental.pallas import tpu_sc as plsc

N_ATOMS = 150000
N_STRUCT = 1000
N_FEAT = 128
N_SPECIES = 4

NC = 2
NS = 16

CHUNK = 125
CHUNKS_PER_TILE = N_ATOMS // (NS * CHUNK)
ROWS_PER_TILE = CHUNK * CHUNKS_PER_TILE
NBUF = 3
NGROUPS = CHUNKS_PER_TILE // NBUF
W = 256
ZROWS = 2 * N_STRUCT // NS


def _sc_body(f0, i0, p0, f1, i1, p1, f2, i2, p2, f3, i3, p3, out_hbm,
             acc_a, acc_b, window, idxbuf, idxpbuf, widx, fbuf, lsem, ssem):
  cid = lax.axis_index("c")
  sid = lax.axis_index("s")
  iota16 = lax.iota(jnp.int32, 16)
  zero16 = jnp.zeros((16,), jnp.float32)

  def zero_window():
    def zrow(r, carry):
      for c in range(N_FEAT // 16):
        window[r, pl.ds(c * 16, 16)] = zero16
      return carry
    lax.fori_loop(0, W, zrow, None)

  zero_window()

  @pl.when(sid < NS // 2)
  def _():
    pltpu.sync_copy(window.at[pl.ds(0, ZROWS)],
                    acc_a.at[pl.ds(sid * ZROWS, ZROWS)])

  @pl.when(sid >= NS // 2)
  def _():
    pltpu.sync_copy(window.at[pl.ds(0, ZROWS)],
                    acc_b.at[pl.ds((sid - NS // 2) * ZROWS, ZROWS)])

  plsc.subcore_barrier()

  def process(f_hbm, idx_hbm, idxp_hbm, acc):
    pltpu.sync_copy(idx_hbm.at[sid], idxbuf)
    pltpu.sync_copy(idxp_hbm.at[sid], idxpbuf)

    lo = jnp.min(idxpbuf[0, pl.ds(0, 16)])
    hi = jnp.max(idxpbuf[CHUNKS_PER_TILE - 1, pl.ds(112, 16)])
    fits = hi - lo + 1 <= W
    lo_splat = jnp.full((16,), lo, jnp.int32)

    def src(k):
      return f_hbm.at[pl.ds(sid * ROWS_PER_TILE + k * CHUNK, CHUNK)]

    @pl.when(fits)
    def _fast():
      col_idx = [iota16 + c * 16 for c in range(N_FEAT // 16)]

      for d in range(W // 128):
        for o in range(0, 128, 16):
          widx[d, pl.ds(o, 16)] = jnp.minimum(
              lo_splat + (d * 128 + o) + iota16, N_STRUCT - 1)

      lane = [jnp.full((16,), j, jnp.int32) for j in range(16)]

      def rows16(k, b, g2, nrows):
        idx_vec = idxpbuf[k, pl.ds(g2 * 16, 16)]
        voff = idx_vec - lo_splat
        rsp_prev = jnp.take_along_axis(voff, lane[0], axis=0)
        acc_v = [fbuf[b, g2 * 16, pl.ds(c * 16, 16)]
                 for c in range(N_FEAT // 16)]
        for j in range(1, nrows):
          rsp = jnp.take_along_axis(voff, lane[j], axis=0)
          bmask = rsp != rsp_prev
          for c in range(N_FEAT // 16):
            plsc.addupdate_scatter(window, [rsp_prev, col_idx[c]], acc_v[c],
                                   mask=bmask)
            x = fbuf[b, g2 * 16 + j, pl.ds(c * 16, 16)]
            acc_v[c] = jnp.where(bmask, x, acc_v[c] + x)
          rsp_prev = rsp
        for c in range(N_FEAT // 16):
          plsc.addupdate_scatter(window, [rsp_prev, col_idx[c]], acc_v[c])

      def tec_chunk(k, b):
        def g2body(g2, carry):
          rows16(k, b, g2, 16)
          return carry
        lax.fori_loop(0, (CHUNK // 16), g2body, None)
        rows16(k, b, CHUNK // 16, CHUNK - 16 * (CHUNK // 16))

      pltpu.async_copy(src(0), fbuf.at[0], lsem.at[0])
      pltpu.async_copy(src(1), fbuf.at[1], lsem.at[1])
      pltpu.async_copy(src(2), fbuf.at[2], lsem.at[2])

      def group(g, carry):
        k0 = 3 * g
        pltpu.make_async_copy(src(k0), fbuf.at[0], lsem.at[0]).wait()
        pltpu.async_copy(fbuf.at[0], acc.at[idxbuf.at[k0]], ssem.at[0],
                         add=True)
        pltpu.make_async_copy(src(k0 + 1), fbuf.at[1], lsem.at[1]).wait()
        pltpu.async_copy(fbuf.at[1], acc.at[idxbuf.at[k0 + 1]], ssem.at[1],
                         add=True)

        pltpu.make_async_copy(src(k0 + 2), fbuf.at[2], lsem.at[2]).wait()
        tec_chunk(k0 + 2, 2)

        @pl.when(g < NGROUPS - 1)
        def _():
          pltpu.async_copy(src(k0 + 5), fbuf.at[2], lsem.at[2])

        pltpu.make_async_copy(fbuf.at[0], acc.at[idxbuf.at[k0]],
                              ssem.at[0]).wait()

        @pl.when(g < NGROUPS - 1)
        def _():
          pltpu.async_copy(src(k0 + 3), fbuf.at[0], lsem.at[0])
        pltpu.make_async_copy(fbuf.at[1], acc.at[idxbuf.at[k0 + 1]],
                              ssem.at[1]).wait()

        @pl.when(g < NGROUPS - 1)
        def _():
          pltpu.async_copy(src(k0 + 4), fbuf.at[1], lsem.at[1])
        return carry

      lax.fori_loop(0, NGROUPS, group, None)

      pltpu.sync_copy(window.at[pl.ds(0, 128)], acc.at[widx.at[0]], add=True)
      pltpu.sync_copy(window.at[pl.ds(128, 128)], acc.at[widx.at[1]],
                      add=True)
      zero_window()

    @pl.when(jnp.logical_not(fits))
    def _slow():
      for b in range(NBUF):
        pltpu.async_copy(src(b), fbuf.at[b], lsem.at[b])

      def group(g, carry):
        for b in range(NBUF):
          k = g * NBUF + b
          pltpu.make_async_copy(src(k), fbuf.at[b], lsem.at[b]).wait()
          pltpu.sync_copy(fbuf.at[b], acc.at[idxbuf.at[k]], add=True)

          @pl.when(g < NGROUPS - 1)
          def _():
            pltpu.async_copy(src(k + NBUF), fbuf.at[b], lsem.at[b])
        return carry

      lax.fori_loop(0, NGROUPS, group, None)

  @pl.when(cid == 0)
  def _():
    process(f0, i0, p0, acc_a)
    process(f1, i1, p1, acc_b)

  @pl.when(cid == 1)
  def _():
    process(f2, i2, p2, acc_a)
    process(f3, i3, p3, acc_b)

  plsc.subcore_barrier()

  @pl.when(sid < NS // 2)
  def _():
    pltpu.sync_copy(acc_a.at[pl.ds(sid * ZROWS, ZROWS)],
                    out_hbm.at[2 * cid, pl.ds(sid * ZROWS, ZROWS)])

  @pl.when(sid >= NS // 2)
  def _():
    pltpu.sync_copy(acc_b.at[pl.ds((sid - NS // 2) * ZROWS, ZROWS)],
                    out_hbm.at[2 * cid + 1, pl.ds((sid - NS // 2) * ZROWS, ZROWS)])


@jax.jit
def _sum_atoms(f0, i0, p0, f1, i1, p1, f2, i2, p2, f3, i3, p3):
  mesh = plsc.VectorSubcoreMesh(
      core_axis_name="c", subcore_axis_name="s", num_cores=NC, num_subcores=NS)
  call = pl.kernel(
      _sc_body,
      out_type=jax.ShapeDtypeStruct((N_SPECIES, N_STRUCT, N_FEAT), jnp.float32),
      mesh=mesh,
      scratch_types=[
          pltpu.VMEM_SHARED((N_STRUCT, N_FEAT), jnp.float32),
          pltpu.VMEM_SHARED((N_STRUCT, N_FEAT), jnp.float32),
          pltpu.VMEM((W, N_FEAT), jnp.float32),
          pltpu.VMEM((CHUNKS_PER_TILE, CHUNK), jnp.int32),
          pltpu.VMEM((CHUNKS_PER_TILE, 128), jnp.int32),
          pltpu.VMEM((W // 128, 128), jnp.int32),
          pltpu.VMEM((NBUF, CHUNK, N_FEAT), jnp.float32),
          pltpu.SemaphoreType.DMA((NBUF,)),
          pltpu.SemaphoreType.DMA((2,)),
      ],
      compiler_params=pltpu.CompilerParams(
          use_tc_tiling_on_sc=False, needs_layout_passes=False),
  )
  return call(f0, i0, p0, f1, i1, p1, f2, i2, p2, f3, i3, p3)


def kernel(features_1, structure_indices_1, features_6, structure_indices_6,
           features_7, structure_indices_7, features_8, structure_indices_8):
  def prep(idx):
    i = idx.astype(jnp.int32).reshape(NS * CHUNKS_PER_TILE, CHUNK)
    ip = jnp.concatenate([i, jnp.broadcast_to(i[:, -1:], (i.shape[0], 3))],
                         axis=1)
    return (i.reshape(NS, CHUNKS_PER_TILE, CHUNK),
            ip.reshape(NS, CHUNKS_PER_TILE, 128))

  i1, p1 = prep(structure_indices_1)
  i6, p6 = prep(structure_indices_6)
  i7, p7 = prep(structure_indices_7)
  i8, p8 = prep(structure_indices_8)
  out = _sum_atoms(features_1, i1, p1, features_6, i6, p6,
                   features_7, i7, p7, features_8, i8, p8)
  return out.transpose(1, 0, 2).reshape(N_STRUCT, N_SPECIES * N_FEAT)


# speedup vs baseline: 1.3800x; 1.3800x over previous
import jax
import jax.numpy as jnp
from jax import lax
from jax.experimental import pallas as pl
from jax.experimental.pallas import tpu as pltpu
from jax.experimental.pallas import tpu_sc as plsc

N_ATOMS = 150000
N_STRUCT = 1000
N_FEAT = 128
N_SPECIES = 4

NC = 2
NS = 16

CHUNK = 125
CHUNKS_PER_TILE = N_ATOMS // (NS * CHUNK)
ROWS_PER_TILE = CHUNK * CHUNKS_PER_TILE
ZROWS = 2 * N_STRUCT // NS
NBUF = 3
NGROUPS = CHUNKS_PER_TILE // NBUF


def _sc_body(f0, i0, f1, i1, f2, i2, f3, i3, out_hbm,
             acc_a, acc_b, zbuf, idxbuf, fbuf, lsem):
  cid = lax.axis_index("c")
  sid = lax.axis_index("s")

  def zstore(i, carry):
    r = i // (N_FEAT // 16)
    c = (i % (N_FEAT // 16)) * 16
    zbuf[r, pl.ds(c, 16)] = jnp.zeros((16,), jnp.float32)
    return carry
  lax.fori_loop(0, ZROWS * (N_FEAT // 16), zstore, None)

  @pl.when(sid < NS // 2)
  def _():
    pltpu.sync_copy(zbuf, acc_a.at[pl.ds(sid * ZROWS, ZROWS)])

  @pl.when(sid >= NS // 2)
  def _():
    pltpu.sync_copy(zbuf, acc_b.at[pl.ds((sid - NS // 2) * ZROWS, ZROWS)])

  plsc.subcore_barrier()

  def process(f_hbm, idx_hbm, acc):
    pltpu.sync_copy(idx_hbm.at[sid], idxbuf)

    def src(k):
      return f_hbm.at[pl.ds(sid * ROWS_PER_TILE + k * CHUNK, CHUNK)]

    for b in range(NBUF):
      pltpu.async_copy(src(b), fbuf.at[b], lsem.at[b])

    def group(g, carry):
      for b in range(NBUF):
        k = g * NBUF + b
        pltpu.make_async_copy(src(k), fbuf.at[b], lsem.at[b]).wait()
        pltpu.sync_copy(fbuf.at[b], acc.at[idxbuf.at[k]], add=True)

        @pl.when(g < NGROUPS - 1)
        def _():
          pltpu.async_copy(src(k + NBUF), fbuf.at[b], lsem.at[b])
      return carry

    lax.fori_loop(0, NGROUPS, group, None)

  @pl.when(cid == 0)
  def _():
    process(f0, i0, acc_a)
    process(f1, i1, acc_b)

  @pl.when(cid == 1)
  def _():
    process(f2, i2, acc_a)
    process(f3, i3, acc_b)

  plsc.subcore_barrier()

  @pl.when(sid < NS // 2)
  def _():
    pltpu.sync_copy(acc_a.at[pl.ds(sid * ZROWS, ZROWS)],
                    out_hbm.at[pl.ds(sid * ZROWS, ZROWS), 2 * cid])

  @pl.when(sid >= NS // 2)
  def _():
    pltpu.sync_copy(
        acc_b.at[pl.ds((sid - NS // 2) * ZROWS, ZROWS)],
        out_hbm.at[pl.ds((sid - NS // 2) * ZROWS, ZROWS), 2 * cid + 1])


@jax.jit
def _sum_atoms(f0, i0, f1, i1, f2, i2, f3, i3):
  mesh = plsc.VectorSubcoreMesh(
      core_axis_name="c", subcore_axis_name="s", num_cores=NC, num_subcores=NS)
  call = pl.kernel(
      _sc_body,
      out_type=jax.ShapeDtypeStruct((N_STRUCT, N_SPECIES, N_FEAT), jnp.float32),
      mesh=mesh,
      scratch_types=[
          pltpu.VMEM_SHARED((N_STRUCT, N_FEAT), jnp.float32),
          pltpu.VMEM_SHARED((N_STRUCT, N_FEAT), jnp.float32),
          pltpu.VMEM((ZROWS, N_FEAT), jnp.float32),
          pltpu.VMEM((CHUNKS_PER_TILE, CHUNK), jnp.int32),
          pltpu.VMEM((NBUF, CHUNK, N_FEAT), jnp.float32),
          pltpu.SemaphoreType.DMA((NBUF,)),
      ],
      compiler_params=pltpu.CompilerParams(use_tc_tiling_on_sc=False),
  )
  return call(f0, i0, f1, i1, f2, i2, f3, i3)


def kernel(features_1, structure_indices_1, features_6, structure_indices_6,
           features_7, structure_indices_7, features_8, structure_indices_8):
  def prep(idx):
    return idx.astype(jnp.int32).reshape(NS, CHUNKS_PER_TILE, CHUNK)

  out = _sum_atoms(
      features_1, prep(structure_indices_1),
      features_6, prep(structure_indices_6),
      features_7, prep(structure_indices_7),
      features_8, prep(structure_indices_8))
  return out.reshape(N_STRUCT, N_SPECIES * N_FEAT)


# speedup vs baseline: 1.4063x; 1.0191x over previous
import jax
import jax.numpy as jnp
from jax import lax
from jax.experimental import pallas as pl
from jax.experimental.pallas import tpu as pltpu
from jax.experimental.pallas import tpu_sc as plsc

N_ATOMS = 150000
N_STRUCT = 1000
N_FEAT = 128
N_SPECIES = 4

NC = 2
NS = 16

CHUNK = 125
CHUNKS_PER_TILE = N_ATOMS // (NS * CHUNK)
ROWS_PER_TILE = CHUNK * CHUNKS_PER_TILE
ZROWS = 2 * N_STRUCT // NS
NBUF = 3
NGROUPS = CHUNKS_PER_TILE // NBUF


def _sc_body(f0, i0, f1, i1, f2, i2, f3, i3, out_hbm,
             acc_a, acc_b, zbuf, idxbuf, fbuf, lsem):
  cid = lax.axis_index("c")
  sid = lax.axis_index("s")

  def src_of(f_hbm, k):
    return f_hbm.at[pl.ds(sid * ROWS_PER_TILE + k * CHUNK, CHUNK)]

  first_f = [f0, f2]
  first_i = [i0, i2]
  for c_val in range(NC):
    @pl.when(cid == c_val)
    def _():
      for b in range(NBUF):
        pltpu.async_copy(src_of(first_f[c_val], b), fbuf.at[b], lsem.at[b])
      pltpu.sync_copy(first_i[c_val].at[sid], idxbuf)

  def zstore(i, carry):
    r = i // (N_FEAT // 16)
    c = (i % (N_FEAT // 16)) * 16
    zbuf[r, pl.ds(c, 16)] = jnp.zeros((16,), jnp.float32)
    return carry
  lax.fori_loop(0, ZROWS * (N_FEAT // 16), zstore, None)

  @pl.when(sid < NS // 2)
  def _():
    pltpu.sync_copy(zbuf, acc_a.at[pl.ds(sid * ZROWS, ZROWS)])

  @pl.when(sid >= NS // 2)
  def _():
    pltpu.sync_copy(zbuf, acc_b.at[pl.ds((sid - NS // 2) * ZROWS, ZROWS)])

  plsc.subcore_barrier()

  def process(f_hbm, idx_hbm, acc, primed=False):
    def src(k):
      return src_of(f_hbm, k)

    if not primed:
      pltpu.sync_copy(idx_hbm.at[sid], idxbuf)
      for b in range(NBUF):
        pltpu.async_copy(src(b), fbuf.at[b], lsem.at[b])

    def group(g, carry):
      for b in range(NBUF):
        k = g * NBUF + b
        pltpu.make_async_copy(src(k), fbuf.at[b], lsem.at[b]).wait()
        pltpu.sync_copy(fbuf.at[b], acc.at[idxbuf.at[k]], add=True)

        @pl.when(g < NGROUPS - 1)
        def _():
          pltpu.async_copy(src(k + NBUF), fbuf.at[b], lsem.at[b])
      return carry

    lax.fori_loop(0, NGROUPS, group, None)

  @pl.when(cid == 0)
  def _():
    process(f0, i0, acc_a, primed=True)
    process(f1, i1, acc_b)

  @pl.when(cid == 1)
  def _():
    process(f2, i2, acc_a, primed=True)
    process(f3, i3, acc_b)

  plsc.subcore_barrier()

  @pl.when(sid < NS // 2)
  def _():
    pltpu.sync_copy(acc_a.at[pl.ds(sid * ZROWS, ZROWS)],
                    out_hbm.at[pl.ds(sid * ZROWS, ZROWS), 2 * cid])

  @pl.when(sid >= NS // 2)
  def _():
    pltpu.sync_copy(
        acc_b.at[pl.ds((sid - NS // 2) * ZROWS, ZROWS)],
        out_hbm.at[pl.ds((sid - NS // 2) * ZROWS, ZROWS), 2 * cid + 1])


@jax.jit
def _sum_atoms(f0, i0, f1, i1, f2, i2, f3, i3):
  mesh = plsc.VectorSubcoreMesh(
      core_axis_name="c", subcore_axis_name="s", num_cores=NC, num_subcores=NS)
  call = pl.kernel(
      _sc_body,
      out_type=jax.ShapeDtypeStruct((N_STRUCT, N_SPECIES, N_FEAT), jnp.float32),
      mesh=mesh,
      scratch_types=[
          pltpu.VMEM_SHARED((N_STRUCT, N_FEAT), jnp.float32),
          pltpu.VMEM_SHARED((N_STRUCT, N_FEAT), jnp.float32),
          pltpu.VMEM((ZROWS, N_FEAT), jnp.float32),
          pltpu.VMEM((CHUNKS_PER_TILE, CHUNK), jnp.int32),
          pltpu.VMEM((NBUF, CHUNK, N_FEAT), jnp.float32),
          pltpu.SemaphoreType.DMA((NBUF,)),
      ],
      compiler_params=pltpu.CompilerParams(use_tc_tiling_on_sc=False),
  )
  return call(f0, i0, f1, i1, f2, i2, f3, i3)


def kernel(features_1, structure_indices_1, features_6, structure_indices_6,
           features_7, structure_indices_7, features_8, structure_indices_8):
  def prep(idx):
    return idx.astype(jnp.int32).reshape(NS, CHUNKS_PER_TILE, CHUNK)

  out = _sum_atoms(
      features_1, prep(structure_indices_1),
      features_6, prep(structure_indices_6),
      features_7, prep(structure_indices_7),
      features_8, prep(structure_indices_8))
  return out.reshape(N_STRUCT, N_SPECIES * N_FEAT)


# speedup vs baseline: 1.4232x; 1.0120x over previous
import jax
import jax.numpy as jnp
from jax import lax
from jax.experimental import pallas as pl
from jax.experimental.pallas import tpu as pltpu
from jax.experimental.pallas import tpu_sc as plsc

N_ATOMS = 150000
N_STRUCT = 1000
N_FEAT = 128
N_SPECIES = 4

NC = 2
NS = 16

CHUNK = 125
CHUNKS_PER_TILE = N_ATOMS // (NS * CHUNK)
ROWS_PER_TILE = CHUNK * CHUNKS_PER_TILE
ZROWS = 2 * N_STRUCT // NS
NBUF = 3
NGROUPS = CHUNKS_PER_TILE // NBUF


def _sc_body(f0, i0, f1, i1, f2, i2, f3, i3, out_hbm,
             acc_a, acc_b, zbuf, idxbuf, idxbuf2, fbuf, lsem):
  cid = lax.axis_index("c")
  sid = lax.axis_index("s")

  def src_of(f_hbm, k):
    return f_hbm.at[pl.ds(sid * ROWS_PER_TILE + k * CHUNK, CHUNK)]

  first_f = [f0, f2]
  first_i = [i0, i2]
  second_i = [i1, i3]
  for c_val in range(NC):
    @pl.when(cid == c_val)
    def _():
      for b in range(NBUF):
        pltpu.async_copy(src_of(first_f[c_val], b), fbuf.at[b], lsem.at[b])
      pltpu.sync_copy(first_i[c_val].at[sid], idxbuf)
      pltpu.sync_copy(second_i[c_val].at[sid], idxbuf2)

  def zstore(i, carry):
    r = i // (N_FEAT // 16)
    c = (i % (N_FEAT // 16)) * 16
    zbuf[r, pl.ds(c, 16)] = jnp.zeros((16,), jnp.float32)
    return carry
  lax.fori_loop(0, ZROWS * (N_FEAT // 16), zstore, None)

  @pl.when(sid < NS // 2)
  def _():
    pltpu.sync_copy(zbuf, acc_a.at[pl.ds(sid * ZROWS, ZROWS)])

  @pl.when(sid >= NS // 2)
  def _():
    pltpu.sync_copy(zbuf, acc_b.at[pl.ds((sid - NS // 2) * ZROWS, ZROWS)])

  plsc.subcore_barrier()

  def process(f_hbm, idx_ref, acc, next_f=None):
    def src(k):
      return src_of(f_hbm, k)

    def group(g, carry):
      for b in range(NBUF):
        k = g * NBUF + b
        pltpu.make_async_copy(src(k), fbuf.at[b], lsem.at[b]).wait()
        pltpu.sync_copy(fbuf.at[b], acc.at[idx_ref.at[k]], add=True)

        @pl.when(g < NGROUPS - 1)
        def _():
          pltpu.async_copy(src(k + NBUF), fbuf.at[b], lsem.at[b])

        if next_f is not None:
          @pl.when(g == NGROUPS - 1)
          def _():
            pltpu.async_copy(src_of(next_f, b), fbuf.at[b], lsem.at[b])
      return carry

    lax.fori_loop(0, NGROUPS, group, None)

  @pl.when(cid == 0)
  def _():
    process(f0, idxbuf, acc_a, next_f=f1)
    process(f1, idxbuf2, acc_b)

  @pl.when(cid == 1)
  def _():
    process(f2, idxbuf, acc_a, next_f=f3)
    process(f3, idxbuf2, acc_b)

  plsc.subcore_barrier()

  @pl.when(sid < NS // 2)
  def _():
    pltpu.sync_copy(acc_a.at[pl.ds(sid * ZROWS, ZROWS)],
                    out_hbm.at[pl.ds(sid * ZROWS, ZROWS), 2 * cid])

  @pl.when(sid >= NS // 2)
  def _():
    pltpu.sync_copy(
        acc_b.at[pl.ds((sid - NS // 2) * ZROWS, ZROWS)],
        out_hbm.at[pl.ds((sid - NS // 2) * ZROWS, ZROWS), 2 * cid + 1])


@jax.jit
def _sum_atoms(f0, i0, f1, i1, f2, i2, f3, i3):
  mesh = plsc.VectorSubcoreMesh(
      core_axis_name="c", subcore_axis_name="s", num_cores=NC, num_subcores=NS)
  call = pl.kernel(
      _sc_body,
      out_type=jax.ShapeDtypeStruct((N_STRUCT, N_SPECIES, N_FEAT), jnp.float32),
      mesh=mesh,
      scratch_types=[
          pltpu.VMEM_SHARED((N_STRUCT, N_FEAT), jnp.float32),
          pltpu.VMEM_SHARED((N_STRUCT, N_FEAT), jnp.float32),
          pltpu.VMEM((ZROWS, N_FEAT), jnp.float32),
          pltpu.VMEM((CHUNKS_PER_TILE, CHUNK), jnp.int32),
          pltpu.VMEM((CHUNKS_PER_TILE, CHUNK), jnp.int32),
          pltpu.VMEM((NBUF, CHUNK, N_FEAT), jnp.float32),
          pltpu.SemaphoreType.DMA((NBUF,)),
      ],
      compiler_params=pltpu.CompilerParams(use_tc_tiling_on_sc=False),
  )
  return call(f0, i0, f1, i1, f2, i2, f3, i3)


def kernel(features_1, structure_indices_1, features_6, structure_indices_6,
           features_7, structure_indices_7, features_8, structure_indices_8):
  def prep(idx):
    return idx.astype(jnp.int32).reshape(NS, CHUNKS_PER_TILE, CHUNK)

  out = _sum_atoms(
      features_1, prep(structure_indices_1),
      features_6, prep(structure_indices_6),
      features_7, prep(structure_indices_7),
      features_8, prep(structure_indices_8))
  return out.reshape(N_STRUCT, N_SPECIES * N_FEAT)
